# baseline (speedup 1.0000x reference)
import functools
import math

import jax
import jax.numpy as jnp
from jax import lax
from jax.experimental import pallas as pl
from jax.experimental.pallas import tpu as pltpu
from jax.experimental.pallas import tpu_sc as plsc

_B = 16384
_D = 16
_PERIOD = 365.25
_NC, _NS = 2, 16
_NW = _NC * _NS
_NTILE = 7813
_RANGE = 245
_WT = 6
_NWIN = 41
_MAXC = _NTILE - _WT
_FLT = 784

_mesh = plsc.VectorSubcoreMesh(core_axis_name="c", subcore_axis_name="s")
_lane = lambda: lax.broadcasted_iota(jnp.int32, (16,), 0)


def _splat(v):
    return jax.lax.broadcast(v, (16,))


def _take(v, i):
    return lax.gather(
        v, i.reshape(16, 1),
        lax.GatherDimensionNumbers(
            offset_dims=(), collapsed_slice_dims=(0,), start_index_map=(0,)),
        slice_sizes=(1,),
        mode=lax.GatherScatterMode.PROMISE_IN_BOUNDS)


@functools.partial(
    pl.kernel,
    mesh=_mesh,
    out_type=jax.ShapeDtypeStruct((_B * 2 * _D,), jnp.float32),
    scratch_types=[
        pltpu.VMEM((_B,), jnp.int32),
        pltpu.VMEM((_FLT + 16,), jnp.int32),
        pltpu.VMEM((_FLT + 16,), jnp.int32),
        pltpu.VMEM((_D, _WT * 128), jnp.float32),
        pltpu.VMEM((_D, _WT * 128), jnp.float32),
        pltpu.VMEM((_D, _WT * 128), jnp.float32),
        pltpu.VMEM((_D, _WT * 128), jnp.float32),
        pltpu.VMEM((_D, _WT * 128), jnp.float32),
        pltpu.VMEM((_D, _WT * 128), jnp.float32),
        pltpu.VMEM(((_FLT + 16) * 2 * _D,), jnp.float32),
        pltpu.VMEM((256,), jnp.float32),
        pltpu.SemaphoreType.DMA,
        pltpu.SemaphoreType.DMA,
        pltpu.SemaphoreType.DMA,
        pltpu.SemaphoreType.DMA,
        pltpu.SemaphoreType.DMA,
        pltpu.SemaphoreType.DMA,
        pltpu.SemaphoreType.DMA,
    ],
    compiler_params=pltpu.CompilerParams(needs_layout_passes=False),
)
def _sc_gather(idx_hbm, a_hbm, b_hbm, o_hbm,
               ids_v, flt_v, wl_v, ba0, bb0, ba1, bb1, ba2, bb2,
               ca_v, dummy_v,
               sa0, sb0, sa1, sb1, sa2, sb2, so):
    wid = lax.axis_index("s") * _NC + lax.axis_index("c")
    lo = wid * _RANGE
    lane = _lane()

    def filt(g_, off):
        v = ids_v[pl.ds(g_ * 16, 16)]
        roff = v - lo * 128
        m = (roff >= 0) & (roff < _RANGE * 128)
        packed = (roff << 14) | (g_ * 16 + lane)
        plsc.store_compressed(
            flt_v.at[pl.ds(jnp.minimum(off, _FLT), 16)], packed, mask=m)
        cnt = plsc.all_reduce_population_count(m)
        return off + cnt[0]

    def fire(w, ba, bb, sa, sb):
        cs = pl.multiple_of(jnp.minimum(lo + w * _WT, _MAXC) * 128, 128)
        pltpu.async_copy(a_hbm.at[:, pl.ds(cs, _WT * 128)], ba, sa)
        pltpu.async_copy(b_hbm.at[:, pl.ds(cs, _WT * 128)], bb, sb)

    def drain(ba, bb, sa, sb):
        pltpu.make_async_copy(a_hbm.at[:, pl.ds(0, _WT * 128)], ba, sa).wait()
        pltpu.make_async_copy(b_hbm.at[:, pl.ds(0, _WT * 128)], bb, sb).wait()

    def extract(w, ba, bb, n, m0):
        cs_t = jnp.minimum(lo + w * _WT, _MAXC)
        sbase = cs_t - lo

        def scan(g_, off):
            p = flt_v[pl.ds(g_ * 16, 16)]
            ro = p >> 14
            m = (ro >= sbase * 128) & (ro < (sbase + _WT) * 128) & (
                g_ * 16 + lane < n)
            plsc.store_compressed(wl_v.at[pl.ds(off, 16)], p, mask=m)
            cnt = plsc.all_reduce_population_count(m)
            return off + cnt[0]

        nw = lax.fori_loop(0, (n + 15) // 16, scan, 0)

        def one(k, m_):
            ch = wl_v[pl.ds((k // 16) * 16, 16)]
            e = _take(ch, _splat(k % 16))
            col = (e >> 14) - sbase * 128
            bpos = e & 16383
            va = plsc.load_gather(ba, [lane, col])
            vb = plsc.load_gather(bb, [lane, col])
            slot = m_ * 32
            dst = _splat(slot) + lane
            plsc.store_scatter(ca_v, [dst], va)
            plsc.store_scatter(ca_v, [dst + 16], vb)
            off = pl.multiple_of(bpos[0] * 32, 32)
            pltpu.async_copy(ca_v.at[pl.ds(slot, 32)], o_hbm.at[pl.ds(off, 32)], so)
            return m_ + 1

        return lax.fori_loop(0, nw, one, m0)

    bufs = ((ba0, bb0, sa0, sb0), (ba1, bb1, sa1, sb1), (ba2, bb2, sa2, sb2))
    for j, (ba, bb, sa, sb) in enumerate(bufs):
        fire(j, ba, bb, sa, sb)

    pltpu.sync_copy(idx_hbm, ids_v)
    n = jnp.minimum(lax.fori_loop(0, _B // 16, filt, 0), _FLT)

    def wbody(k, m_):
        for j, (ba, bb, sa, sb) in enumerate(bufs):
            w = 3 * k + j
            drain(ba, bb, sa, sb)
            m_ = extract(w, ba, bb, n, m_)

            @pl.when(w + 3 < _NWIN)
            def _():
                fire(w + 3, ba, bb, sa, sb)

        return m_

    m = lax.fori_loop(0, _NWIN // 3, wbody, 0)
    drain(ba0, bb0, sa0, sb0)
    m = extract(_NWIN - 2, ba0, bb0, n, m)
    drain(ba1, bb1, sa1, sb1)
    m = extract(_NWIN - 1, ba1, bb1, n, m)

    def drain8(q, _):
        pltpu.make_async_copy(o_hbm.at[pl.ds(0, 256)], dummy_v, so).wait()
        return 0

    def drain1(q, _):
        pltpu.make_async_copy(
            o_hbm.at[pl.ds(0, 32)], dummy_v.at[pl.ds(0, 32)], so).wait()
        return 0

    lax.fori_loop(0, m // 8, drain8, 0)
    lax.fori_loop(0, m % 8, drain1, 0)


_ROWS = _B * 2 * _D // 128
_BR = 512


def _tc_combine(t_ref, c_ref, o_ref):
    li = lax.broadcasted_iota(jnp.int32, (128, 4), 0)
    ji = lax.broadcasted_iota(jnp.int32, (128, 4), 1)
    s = (li // 32 == ji).astype(jnp.float32)
    lit = lax.broadcasted_iota(jnp.int32, (4, 128), 1)
    jit = lax.broadcasted_iota(jnp.int32, (4, 128), 0)
    st = (lit // 32 == jit).astype(jnp.float32)
    t = jnp.dot(t_ref[...], st, preferred_element_type=jnp.float32)
    ln = lax.broadcasted_iota(jnp.int32, (_BR, 128), 1)
    n = (ln % 16 + 1).astype(jnp.float32)
    x = (2.0 * math.pi / _PERIOD) * (n * t)
    feat = jnp.where(ln % 32 < 16, jnp.cos(x), jnp.sin(x))
    acc = feat * c_ref[...]
    o_ref[...] = jnp.dot(acc, s, preferred_element_type=jnp.float32)


def kernel(t, id, a_table, b_table):
    idx = id.reshape(-1).astype(jnp.int32)
    rc = _sc_gather(idx, a_table.T, b_table.T)
    out2 = pl.pallas_call(
        _tc_combine,
        grid=(_ROWS // _BR,),
        in_specs=[
            pl.BlockSpec((_BR, 4), lambda i: (i, 0)),
            pl.BlockSpec((_BR, 128), lambda i: (i, 0)),
        ],
        out_specs=pl.BlockSpec((_BR, 4), lambda i: (i, 0)),
        out_shape=jax.ShapeDtypeStruct((_ROWS, 4), jnp.float32),
    )(t.reshape(_ROWS, 4), rc.reshape(_ROWS, 128))
    return out2.reshape(_B, 1)
